# baseline (speedup 1.0000x reference)
import functools

import jax
import jax.numpy as jnp
from jax import lax
from jax.experimental import pallas as pl
from jax.experimental.pallas import tpu as pltpu
from jax.experimental.pallas import tpu_sc as plsc

N_CAT = 26
N_CONT = 13
VOCAB = 100000
DIM = 16
B = 16384
N_FIELD = N_CAT + N_CONT

NC = 2
NS = 16
NW = NC * NS
ROWS_PER_W = B // NW
CHUNK = 64
N_CHUNKS = ROWS_PER_W // CHUNK


def _tc_prep_body(x_ref, lnb_ref, linw_ref, linb_ref, gidx_ref, cont_ref):
    x = x_ref[...]
    offs = lax.broadcasted_iota(jnp.int32, (B, N_CAT), 1) * VOCAB
    gidx_ref[...] = x[:, :N_CAT].astype(jnp.int32) + offs
    c = lnb_ref[...][:, 0:1] * linw_ref[...] + linb_ref[...]
    cont_ref[...] = jnp.broadcast_to(c[None], (CHUNK, N_CONT, DIM))


def _sc_body(table_ref, gidx_ref, cont_ref, out_ref, idx_v, rows_v, sem):
    wid = lax.axis_index("s") * NC + lax.axis_index("c")
    base = wid * ROWS_PER_W
    pltpu.sync_copy(cont_ref, rows_v.at[:, pl.ds(N_CAT, N_CONT)])

    @pl.loop(0, N_CHUNKS)
    def _chunk(c):
        row0 = base + c * CHUNK
        pltpu.sync_copy(gidx_ref.at[pl.ds(row0, CHUNK)], idx_v)

        @pl.loop(0, CHUNK)
        def _fire(lb):
            pltpu.async_copy(
                table_ref.at[idx_v.at[lb]],
                rows_v.at[lb, pl.ds(0, N_CAT)],
                sem,
            )

        @pl.loop(0, CHUNK)
        def _drain(lb):
            pltpu.make_async_copy(
                table_ref.at[idx_v.at[lb]],
                rows_v.at[lb, pl.ds(0, N_CAT)],
                sem,
            ).wait()

        pltpu.sync_copy(rows_v, out_ref.at[pl.ds(row0, CHUNK)])


@jax.jit
def kernel(X, cat_tables, ln_w, ln_b, lin_w, lin_b):
    del ln_w
    flat_table = cat_tables.reshape(N_CAT * VOCAB, DIM)
    gidx, cont_rep = pl.pallas_call(
        _tc_prep_body,
        out_shape=(
            jax.ShapeDtypeStruct((B, N_CAT), jnp.int32),
            jax.ShapeDtypeStruct((CHUNK, N_CONT, DIM), jnp.float32),
        ),
    )(X, ln_b, lin_w, lin_b)

    sc = pl.kernel(
        _sc_body,
        out_type=jax.ShapeDtypeStruct((B, N_FIELD, DIM), jnp.float32),
        mesh=plsc.VectorSubcoreMesh(core_axis_name="c", subcore_axis_name="s"),
        compiler_params=pltpu.CompilerParams(use_tc_tiling_on_sc=False),
        scratch_types=[
            pltpu.VMEM((CHUNK, N_CAT), jnp.int32),
            pltpu.VMEM((CHUNK, N_FIELD, DIM), jnp.float32),
            pltpu.SemaphoreType.DMA,
        ],
    )
    out3 = sc(flat_table, gidx, cont_rep)
    return out3.reshape(B, N_CAT * DIM + N_CONT * DIM)


# speedup vs baseline: 2.2358x; 2.2358x over previous
import functools

import jax
import jax.numpy as jnp
from jax import lax
from jax.experimental import pallas as pl
from jax.experimental.pallas import tpu as pltpu
from jax.experimental.pallas import tpu_sc as plsc

N_CAT = 26
N_CONT = 13
VOCAB = 100000
DIM = 16
B = 16384
N_FIELD = N_CAT + N_CONT
C_PAD = 32

NC = 2
NS = 16
NW = NC * NS
ROWS_PER_W = B // NW
CHUNK = 64
N_CHUNKS = ROWS_PER_W // CHUNK

VB = 512


def _tc_transpose_body(tab_ref, out_ref):
    x = tab_ref[...]
    y = jnp.concatenate(
        [x.T, jnp.zeros((VB, C_PAD * DIM - N_CAT * DIM), jnp.float32)], axis=1
    )
    out_ref[...] = y.reshape(4 * VB, 128)


def _tc_prep_body(x_ref, lnb_ref, linw_ref, linb_ref, gidx_ref, cont_ref):
    x = x_ref[...]
    offs = lax.broadcasted_iota(jnp.int32, (B, N_CAT), 1)
    gidx_ref[...] = x[:, :N_CAT].astype(jnp.int32) * C_PAD + offs
    c = lnb_ref[...][:, 0:1] * linw_ref[...] + linb_ref[...]
    cont_ref[...] = jnp.broadcast_to(c[None], (CHUNK, N_CONT, DIM))


def _sc_body(table_ref, gidx_ref, cont_ref, out_ref, idx_v, rows_v, sem):
    wid = lax.axis_index("s") * NC + lax.axis_index("c")
    base = wid * ROWS_PER_W
    pltpu.sync_copy(cont_ref, rows_v.at[:, pl.ds(N_CAT, N_CONT)])

    @pl.loop(0, N_CHUNKS)
    def _chunk(c):
        row0 = base + c * CHUNK
        pltpu.sync_copy(gidx_ref.at[pl.ds(row0, CHUNK)], idx_v)

        @pl.loop(0, CHUNK)
        def _fire(lb):
            pltpu.async_copy(
                table_ref.at[idx_v.at[lb]],
                rows_v.at[lb, pl.ds(0, N_CAT)],
                sem,
            )

        @pl.loop(0, CHUNK)
        def _drain(lb):
            pltpu.make_async_copy(
                table_ref.at[idx_v.at[lb]],
                rows_v.at[lb, pl.ds(0, N_CAT)],
                sem,
            ).wait()

        pltpu.sync_copy(rows_v, out_ref.at[pl.ds(row0, CHUNK)])


@jax.jit
def kernel(X, cat_tables, ln_w, ln_b, lin_w, lin_b):
    del ln_w
    table_cv = cat_tables.transpose(0, 2, 1).reshape(N_CAT * DIM, VOCAB)
    n_vb = (VOCAB + VB - 1) // VB
    flat512 = pl.pallas_call(
        _tc_transpose_body,
        grid=(n_vb,),
        in_specs=[pl.BlockSpec((N_CAT * DIM, VB), lambda g: (0, g))],
        out_specs=pl.BlockSpec((4 * VB, 128), lambda g: (g, 0)),
        out_shape=jax.ShapeDtypeStruct((4 * VOCAB, 128), jnp.float32),
    )(table_cv)
    flat_table = flat512.reshape(VOCAB * C_PAD, DIM)

    gidx, cont_rep = pl.pallas_call(
        _tc_prep_body,
        out_shape=(
            jax.ShapeDtypeStruct((B, N_CAT), jnp.int32),
            jax.ShapeDtypeStruct((CHUNK, N_CONT, DIM), jnp.float32),
        ),
    )(X, ln_b, lin_w, lin_b)

    sc = pl.kernel(
        _sc_body,
        out_type=jax.ShapeDtypeStruct((B, N_FIELD, DIM), jnp.float32),
        mesh=plsc.VectorSubcoreMesh(core_axis_name="c", subcore_axis_name="s"),
        compiler_params=pltpu.CompilerParams(use_tc_tiling_on_sc=False),
        scratch_types=[
            pltpu.VMEM((CHUNK, N_CAT), jnp.int32),
            pltpu.VMEM((CHUNK, N_FIELD, DIM), jnp.float32),
            pltpu.SemaphoreType.DMA,
        ],
    )
    out3 = sc(flat_table, gidx, cont_rep)
    return out3.reshape(B, N_CAT * DIM + N_CONT * DIM)


# speedup vs baseline: 2.5653x; 1.1474x over previous
import functools

import jax
import jax.numpy as jnp
from jax import lax
from jax.experimental import pallas as pl
from jax.experimental.pallas import tpu as pltpu
from jax.experimental.pallas import tpu_sc as plsc

N_CAT = 26
N_CONT = 13
VOCAB = 100000
DIM = 16
B = 16384
N_FIELD = N_CAT + N_CONT
C_PAD = 32

NC = 2
NS = 16
NW = NC * NS
ROWS_PER_W = B // NW
CHUNK = 64
N_CHUNKS = ROWS_PER_W // CHUNK

VB = 2048


def _tc_transpose_body(tab_ref, out_ref):
    x = tab_ref[...]
    y = jnp.concatenate(
        [x.T, jnp.zeros((VB, C_PAD * DIM - N_CAT * DIM), jnp.float32)], axis=1
    )
    out_ref[...] = y.reshape(4 * VB, 128)


def _tc_prep_body(x_ref, lnb_ref, linw_ref, linb_ref, gidx_ref, cont_ref):
    x = x_ref[...]
    offs = lax.broadcasted_iota(jnp.int32, (B, N_CAT), 1)
    gidx_ref[...] = x[:, :N_CAT].astype(jnp.int32) * C_PAD + offs
    c = lnb_ref[...][:, 0:1] * linw_ref[...] + linb_ref[...]
    cont_ref[...] = jnp.broadcast_to(c[None], (CHUNK, N_CONT, DIM))


def _sc_body(table_ref, gidx_ref, cont_ref, out_ref, idx_v, rows_v, sem):
    wid = lax.axis_index("s") * NC + lax.axis_index("c")
    base = wid * ROWS_PER_W
    pltpu.sync_copy(cont_ref, rows_v.at[:, pl.ds(N_CAT, N_CONT)])

    @pl.loop(0, N_CHUNKS)
    def _chunk(c):
        row0 = base + c * CHUNK
        pltpu.sync_copy(gidx_ref.at[pl.ds(row0, CHUNK)], idx_v)

        @pl.loop(0, CHUNK)
        def _fire(lb):
            pltpu.async_copy(
                table_ref.at[idx_v.at[lb]],
                rows_v.at[lb, pl.ds(0, N_CAT)],
                sem,
            )

        @pl.loop(0, CHUNK)
        def _drain(lb):
            pltpu.make_async_copy(
                table_ref.at[idx_v.at[lb]],
                rows_v.at[lb, pl.ds(0, N_CAT)],
                sem,
            ).wait()

        pltpu.sync_copy(rows_v, out_ref.at[pl.ds(row0, CHUNK)])


@jax.jit
def kernel(X, cat_tables, ln_w, ln_b, lin_w, lin_b):
    del ln_w
    table_cv = cat_tables.transpose(0, 2, 1).reshape(N_CAT * DIM, VOCAB)
    n_vb = (VOCAB + VB - 1) // VB
    flat512 = pl.pallas_call(
        _tc_transpose_body,
        grid=(n_vb,),
        in_specs=[pl.BlockSpec((N_CAT * DIM, VB), lambda g: (0, g))],
        out_specs=pl.BlockSpec((4 * VB, 128), lambda g: (g, 0)),
        out_shape=jax.ShapeDtypeStruct((4 * VOCAB, 128), jnp.float32),
    )(table_cv)
    flat_table = flat512.reshape(VOCAB * C_PAD, DIM)

    gidx, cont_rep = pl.pallas_call(
        _tc_prep_body,
        out_shape=(
            jax.ShapeDtypeStruct((B, N_CAT), jnp.int32),
            jax.ShapeDtypeStruct((CHUNK, N_CONT, DIM), jnp.float32),
        ),
    )(X, ln_b, lin_w, lin_b)

    sc = pl.kernel(
        _sc_body,
        out_type=jax.ShapeDtypeStruct((B, N_FIELD, DIM), jnp.float32),
        mesh=plsc.VectorSubcoreMesh(core_axis_name="c", subcore_axis_name="s"),
        compiler_params=pltpu.CompilerParams(use_tc_tiling_on_sc=False),
        scratch_types=[
            pltpu.VMEM((CHUNK, N_CAT), jnp.int32),
            pltpu.VMEM((CHUNK, N_FIELD, DIM), jnp.float32),
            pltpu.SemaphoreType.DMA,
        ],
    )
    out3 = sc(flat_table, gidx, cont_rep)
    return out3.reshape(B, N_CAT * DIM + N_CONT * DIM)
